# baseline (speedup 1.0000x reference)
import functools

import jax
import jax.numpy as jnp
from jax import lax
from jax.experimental import pallas as pl
from jax.experimental.pallas import tpu as pltpu
from jax.experimental.pallas import tpu_sc as plsc

D = 128
H = 128
W = 128
BE = 2000
BN = 2000
SC_CH = 400
NW = 32


def _proj_body(x_ref, w_ref, o_ref):
    o_ref[...] = jnp.dot(x_ref[...], w_ref[...],
                         preferred_element_type=jnp.float32)


def _project(x, w):
    n = x.shape[0]
    return pl.pallas_call(
        _proj_body,
        grid=(n // BN,),
        in_specs=[pl.BlockSpec((BN, D), lambda i: (i, 0)),
                  pl.BlockSpec((D, H), lambda i: (0, 0))],
        out_specs=pl.BlockSpec((BN, H), lambda i: (i, 0)),
        out_shape=jax.ShapeDtypeStruct((n, H), jnp.float32),
    )(x, w)


def _sc_gather(table, idx):
    e = idx.shape[0]
    per_w = e // NW
    mesh = plsc.VectorSubcoreMesh(core_axis_name="c", subcore_axis_name="s")

    @functools.partial(
        pl.kernel, mesh=mesh,
        out_type=jax.ShapeDtypeStruct((e, H), jnp.float32),
        scratch_types=[pltpu.VMEM((SC_CH,), jnp.int32),
                       pltpu.VMEM((SC_CH, H), jnp.float32),
                       pltpu.SemaphoreType.DMA],
    )
    def k(table_hbm, idx_hbm, out_hbm, idx_v, rows_v, sem):
        wid = lax.axis_index("s") * 2 + lax.axis_index("c")
        base = wid * per_w

        @pl.loop(0, per_w, step=SC_CH)
        def _(j):
            pltpu.sync_copy(idx_hbm.at[pl.ds(base + j, SC_CH)], idx_v)
            pltpu.async_copy(table_hbm.at[idx_v], rows_v, sem).wait()
            pltpu.sync_copy(rows_v, out_hbm.at[pl.ds(base + j, SC_CH)])

    return k(table, idx)


def _edge_body(m2g_ref, gsrc_ref, dst_ref, pdst_ref, we1_ref, we2_ref,
               b1_ref, b2_ref, ge_ref, be_ref, agg_ref):
    i = pl.program_id(0)

    @pl.when(i == 0)
    def _():
        agg_ref[...] = jnp.zeros_like(agg_ref)

    dst = dst_ref[0]
    w0 = (dst[0, 0] // 8) * 8
    nwin = (dst[0, BE - 1] - w0) // W + 1

    row_ids = lax.broadcasted_iota(jnp.int32, (W, BE), 0)

    def exp_step(k, acc):
        w = w0 + k * W
        oh = jnp.where(row_ids + w == dst, 1.0, 0.0).astype(jnp.bfloat16)
        rows = pdst_ref[pl.ds(w, W), :].astype(jnp.bfloat16)
        return acc + lax.dot_general(
            oh, rows, (((0,), (0,)), ((), ())),
            preferred_element_type=jnp.float32)

    pdst_exp = lax.fori_loop(0, nwin, exp_step,
                             jnp.zeros((BE, H), jnp.float32))

    h = (jnp.dot(m2g_ref[...].astype(jnp.bfloat16),
                 we1_ref[...].astype(jnp.bfloat16),
                 preferred_element_type=jnp.float32)
         + gsrc_ref[...] + pdst_exp + b1_ref[...])
    h = h * jax.nn.sigmoid(h)
    e = jnp.dot(h.astype(jnp.bfloat16), we2_ref[...].astype(jnp.bfloat16),
                preferred_element_type=jnp.float32) + b2_ref[...]
    mu = jnp.mean(e, axis=-1, keepdims=True)
    var = jnp.mean((e - mu) * (e - mu), axis=-1, keepdims=True)
    ef = (e - mu) * lax.rsqrt(var + 1e-5) * ge_ref[...] + be_ref[...]
    ef16 = ef.astype(jnp.bfloat16)

    def agg_step(k, _):
        w = w0 + k * W
        oh = jnp.where(row_ids + w == dst, 1.0, 0.0).astype(jnp.bfloat16)
        contrib = jnp.dot(oh, ef16, preferred_element_type=jnp.float32)
        agg_ref[pl.ds(w, W), :] += contrib
        return 0

    lax.fori_loop(0, nwin, agg_step, 0)


def _edge_agg(m2g, gsrc, dst3, pdst_pad, we1, we2, b1, b2, ge, betae,
              ng_pad, ne):
    nb = ne // BE
    return pl.pallas_call(
        _edge_body,
        grid=(nb,),
        in_specs=[
            pl.BlockSpec((BE, D), lambda i: (i, 0)),
            pl.BlockSpec((BE, H), lambda i: (i, 0)),
            pl.BlockSpec((1, 1, BE), lambda i: (i, 0, 0)),
            pl.BlockSpec((ng_pad, H), lambda i: (0, 0)),
            pl.BlockSpec((D, H), lambda i: (0, 0)),
            pl.BlockSpec((H, D), lambda i: (0, 0)),
            pl.BlockSpec((1, H), lambda i: (0, 0)),
            pl.BlockSpec((1, D), lambda i: (0, 0)),
            pl.BlockSpec((1, D), lambda i: (0, 0)),
            pl.BlockSpec((1, D), lambda i: (0, 0)),
        ],
        out_specs=pl.BlockSpec((ng_pad, D), lambda i: (0, 0)),
        out_shape=jax.ShapeDtypeStruct((ng_pad, D), jnp.float32),
    )(m2g, gsrc, dst3, pdst_pad, we1, we2, b1, b2, ge, betae)


def _node_body(agg_ref, gf_ref, wn1_ref, wn2_ref, bn1_ref, bn2_ref,
               gn_ref, betan_ref, o_ref):
    gf = gf_ref[...]
    nh = (jnp.dot(agg_ref[...], wn1_ref[0:D, :],
                  preferred_element_type=jnp.float32)
          + jnp.dot(gf, wn1_ref[D:2 * D, :],
                    preferred_element_type=jnp.float32)
          + bn1_ref[...])
    nh = nh * jax.nn.sigmoid(nh)
    y = jnp.dot(nh, wn2_ref[...], preferred_element_type=jnp.float32) \
        + bn2_ref[...]
    mu = jnp.mean(y, axis=-1, keepdims=True)
    var = jnp.mean((y - mu) * (y - mu), axis=-1, keepdims=True)
    o_ref[...] = ((y - mu) * lax.rsqrt(var + 1e-5) * gn_ref[...]
                  + betan_ref[...] + gf)


def _node_mlp(agg, gf, wn1, wn2, bn1, bn2, gn, betan):
    ng = gf.shape[0]
    return pl.pallas_call(
        _node_body,
        grid=(ng // BN,),
        in_specs=[
            pl.BlockSpec((BN, D), lambda i: (i, 0)),
            pl.BlockSpec((BN, D), lambda i: (i, 0)),
            pl.BlockSpec((2 * D, H), lambda i: (0, 0)),
            pl.BlockSpec((H, D), lambda i: (0, 0)),
            pl.BlockSpec((1, H), lambda i: (0, 0)),
            pl.BlockSpec((1, D), lambda i: (0, 0)),
            pl.BlockSpec((1, D), lambda i: (0, 0)),
            pl.BlockSpec((1, D), lambda i: (0, 0)),
        ],
        out_specs=pl.BlockSpec((BN, D), lambda i: (i, 0)),
        out_shape=jax.ShapeDtypeStruct((ng, D), jnp.float32),
    )(agg, gf, wn1, wn2, bn1, bn2, gn, betan)


def kernel(m2g_efeat, grid_nfeat, mesh_nfeat, src_idx, dst_idx, We1, Ws1,
           Wd1, b1, We2, b2, ge, betae, Wn1, bn1, Wn2, bn2, gn, betan):
    ne = m2g_efeat.shape[0]
    ng = grid_nfeat.shape[0]
    ng_pad = ng + W

    p_src = _project(mesh_nfeat, Ws1)
    p_dst = _project(grid_nfeat, Wd1)
    p_dst_pad = jnp.pad(p_dst, ((0, W), (0, 0)))

    gsrc = _sc_gather(p_src, src_idx)

    dst3 = dst_idx.reshape(ne // BE, 1, BE)
    agg_pad = _edge_agg(
        m2g_efeat, gsrc, dst3, p_dst_pad, We1, We2,
        b1.reshape(1, H), b2.reshape(1, D),
        ge.reshape(1, D), betae.reshape(1, D), ng_pad, ne)
    agg = agg_pad[:ng]

    return _node_mlp(agg, grid_nfeat, Wn1, Wn2,
                     bn1.reshape(1, H), bn2.reshape(1, D),
                     gn.reshape(1, D), betan.reshape(1, D))


# speedup vs baseline: 1.2239x; 1.2239x over previous
import functools

import jax
import jax.numpy as jnp
from jax import lax
from jax.experimental import pallas as pl
from jax.experimental.pallas import tpu as pltpu
from jax.experimental.pallas import tpu_sc as plsc

D = 128
H = 128
W = 128
BE = 2000
BN = 2000
NCHUNK = 5
SC_CH = 400
NW = 32


def _proj_body(x_ref, w_ref, o_ref):
    o_ref[...] = jnp.dot(x_ref[...], w_ref[...],
                         preferred_element_type=jnp.float32)


def _project(x, w):
    n = x.shape[0]
    return pl.pallas_call(
        _proj_body,
        grid=(n // BN,),
        in_specs=[pl.BlockSpec((BN, D), lambda i: (i, 0)),
                  pl.BlockSpec((D, H), lambda i: (0, 0))],
        out_specs=pl.BlockSpec((BN, H), lambda i: (i, 0)),
        out_shape=jax.ShapeDtypeStruct((n, H), jnp.float32),
    )(x, w)


def _sc_gather(table, idx, off, ce):
    per_w = ce // NW
    mesh = plsc.VectorSubcoreMesh(core_axis_name="c", subcore_axis_name="s")

    @functools.partial(
        pl.kernel, mesh=mesh,
        out_type=jax.ShapeDtypeStruct((ce, H), jnp.float32),
        scratch_types=[pltpu.VMEM((SC_CH,), jnp.int32),
                       pltpu.VMEM((SC_CH, H), jnp.float32),
                       pltpu.SemaphoreType.DMA],
    )
    def k(table_hbm, idx_hbm, out_hbm, idx_v, rows_v, sem):
        wid = lax.axis_index("s") * 2 + lax.axis_index("c")
        base = wid * per_w

        @pl.loop(0, per_w, step=SC_CH)
        def _(j):
            pltpu.sync_copy(idx_hbm.at[pl.ds(off + base + j, SC_CH)], idx_v)
            pltpu.async_copy(table_hbm.at[idx_v], rows_v, sem).wait()
            pltpu.sync_copy(rows_v, out_hbm.at[pl.ds(base + j, SC_CH)])

    return k(table, idx)


def _edge_body(m2g_ref, gsrc_ref, dst_ref, pdst_ref, we1_ref, we2_ref,
               b1_ref, b2_ref, ge_ref, be_ref, agg_ref):
    i = pl.program_id(0)

    @pl.when(i == 0)
    def _():
        agg_ref[...] = jnp.zeros_like(agg_ref)

    dst = dst_ref[0]
    w0 = (dst[0, 0] // 8) * 8
    nwin = (dst[0, BE - 1] - w0) // W + 1

    row_ids = lax.broadcasted_iota(jnp.int32, (W, BE), 0)

    def exp_step(k, acc):
        w = w0 + k * W
        oh = jnp.where(row_ids + w == dst, 1.0, 0.0)
        rows = pdst_ref[pl.ds(w, W), :]
        return acc + lax.dot_general(
            oh, rows, (((0,), (0,)), ((), ())),
            preferred_element_type=jnp.float32)

    pdst_exp = lax.fori_loop(0, nwin, exp_step,
                             jnp.zeros((BE, H), jnp.float32))

    h = (jnp.dot(m2g_ref[...], we1_ref[...],
                 preferred_element_type=jnp.float32)
         + gsrc_ref[...] + pdst_exp + b1_ref[...])
    h = h * jax.nn.sigmoid(h)
    e = jnp.dot(h, we2_ref[...], preferred_element_type=jnp.float32) \
        + b2_ref[...]
    mu = jnp.mean(e, axis=-1, keepdims=True)
    var = jnp.mean((e - mu) * (e - mu), axis=-1, keepdims=True)
    ef = (e - mu) * lax.rsqrt(var + 1e-5) * ge_ref[...] + be_ref[...]

    def agg_step(k, _):
        w = w0 + k * W
        oh = jnp.where(row_ids + w == dst, 1.0, 0.0)
        contrib = jnp.dot(oh, ef, preferred_element_type=jnp.float32)
        agg_ref[pl.ds(w, W), :] += contrib
        return 0

    lax.fori_loop(0, nwin, agg_step, 0)


def _edge_agg(m2g, gsrc, dst3, pdst_pad, we1, we2, b1, b2, ge, betae,
              ng_pad, blk_off, nblk):
    return pl.pallas_call(
        _edge_body,
        grid=(nblk,),
        in_specs=[
            pl.BlockSpec((BE, D), lambda i: (blk_off + i, 0)),
            pl.BlockSpec((BE, H), lambda i: (i, 0)),
            pl.BlockSpec((1, 1, BE), lambda i: (blk_off + i, 0, 0)),
            pl.BlockSpec((ng_pad, H), lambda i: (0, 0)),
            pl.BlockSpec((D, H), lambda i: (0, 0)),
            pl.BlockSpec((H, D), lambda i: (0, 0)),
            pl.BlockSpec((1, H), lambda i: (0, 0)),
            pl.BlockSpec((1, D), lambda i: (0, 0)),
            pl.BlockSpec((1, D), lambda i: (0, 0)),
            pl.BlockSpec((1, D), lambda i: (0, 0)),
        ],
        out_specs=pl.BlockSpec((ng_pad, D), lambda i: (0, 0)),
        out_shape=jax.ShapeDtypeStruct((ng_pad, D), jnp.float32),
    )(m2g, gsrc, dst3, pdst_pad, we1, we2, b1, b2, ge, betae)


def _node_body(*refs):
    agg_refs = refs[:NCHUNK]
    (gf_ref, wn1_ref, wn2_ref, bn1_ref, bn2_ref, gn_ref, betan_ref,
     o_ref) = refs[NCHUNK:]
    agg = agg_refs[0][...]
    for r in agg_refs[1:]:
        agg = agg + r[...]
    gf = gf_ref[...]
    nh = (jnp.dot(agg, wn1_ref[0:D, :], preferred_element_type=jnp.float32)
          + jnp.dot(gf, wn1_ref[D:2 * D, :],
                    preferred_element_type=jnp.float32)
          + bn1_ref[...])
    nh = nh * jax.nn.sigmoid(nh)
    y = jnp.dot(nh, wn2_ref[...], preferred_element_type=jnp.float32) \
        + bn2_ref[...]
    mu = jnp.mean(y, axis=-1, keepdims=True)
    var = jnp.mean((y - mu) * (y - mu), axis=-1, keepdims=True)
    o_ref[...] = ((y - mu) * lax.rsqrt(var + 1e-5) * gn_ref[...]
                  + betan_ref[...] + gf)


def _node_mlp(aggs, gf, wn1, wn2, bn1, bn2, gn, betan):
    ng = gf.shape[0]
    agg_specs = [pl.BlockSpec((BN, D), lambda i: (i, 0)) for _ in aggs]
    return pl.pallas_call(
        _node_body,
        grid=(ng // BN,),
        in_specs=agg_specs + [
            pl.BlockSpec((BN, D), lambda i: (i, 0)),
            pl.BlockSpec((2 * D, H), lambda i: (0, 0)),
            pl.BlockSpec((H, D), lambda i: (0, 0)),
            pl.BlockSpec((1, H), lambda i: (0, 0)),
            pl.BlockSpec((1, D), lambda i: (0, 0)),
            pl.BlockSpec((1, D), lambda i: (0, 0)),
            pl.BlockSpec((1, D), lambda i: (0, 0)),
        ],
        out_specs=pl.BlockSpec((BN, D), lambda i: (i, 0)),
        out_shape=jax.ShapeDtypeStruct((ng, D), jnp.float32),
    )(*aggs, gf, wn1, wn2, bn1, bn2, gn, betan)


def kernel(m2g_efeat, grid_nfeat, mesh_nfeat, src_idx, dst_idx, We1, Ws1,
           Wd1, b1, We2, b2, ge, betae, Wn1, bn1, Wn2, bn2, gn, betan):
    ne = m2g_efeat.shape[0]
    ng = grid_nfeat.shape[0]
    ng_pad = ng + W
    ce = ne // NCHUNK

    p_src = _project(mesh_nfeat, Ws1)
    p_dst = _project(grid_nfeat, Wd1)
    p_dst_pad = jnp.pad(p_dst, ((0, W), (0, 0)))

    dst3 = dst_idx.reshape(ne // BE, 1, BE)
    b1r = b1.reshape(1, H)
    b2r = b2.reshape(1, D)
    ger = ge.reshape(1, D)
    betr = betae.reshape(1, D)

    aggs = []
    for c in range(NCHUNK):
        gsrc_c = _sc_gather(p_src, src_idx, c * ce, ce)
        aggs.append(_edge_agg(
            m2g_efeat, gsrc_c, dst3, p_dst_pad, We1, We2,
            b1r, b2r, ger, betr, ng_pad,
            blk_off=c * (ce // BE), nblk=ce // BE))

    return _node_mlp(aggs, grid_nfeat, Wn1, Wn2,
                     bn1.reshape(1, H), bn2.reshape(1, D),
                     gn.reshape(1, D), betan.reshape(1, D))
